# baseline (speedup 1.0000x reference)
import functools

import jax
import jax.numpy as jnp
from jax import lax
from jax.experimental import pallas as pl
from jax.experimental.pallas import tpu as pltpu
from jax.experimental.pallas import tpu_sc as plsc

B = 4096
F_NUM = 13
N_CAT = 26
VOCAB = 100000
E = 32
P = 32
OUT = P + N_CAT * E

NC = 2
NS = 16
NW = NC * NS
BPW = B // NW


def _sc_encoder(gidx, numeric, table_flat, W, b):
    mesh = plsc.VectorSubcoreMesh(core_axis_name="c", subcore_axis_name="s")

    @functools.partial(
        pl.kernel,
        out_type=jax.ShapeDtypeStruct((B, OUT), jnp.float32),
        scratch_types=[
            pltpu.VMEM((N_CAT, BPW), jnp.int32),
            pltpu.VMEM((N_CAT, BPW, E), jnp.float32),
            pltpu.VMEM((BPW * F_NUM + 16,), jnp.float32),
            pltpu.VMEM((F_NUM, P), jnp.float32),
            pltpu.VMEM((P,), jnp.float32),
            pltpu.VMEM((BPW, P), jnp.float32),
            pltpu.SemaphoreType.DMA,
        ],
        mesh=mesh,
        compiler_params=pltpu.CompilerParams(use_tc_tiling_on_sc=False),
    )
    def enc(gidx_hbm, num_hbm, tab_hbm, w_hbm, b_hbm, out_hbm,
            idx_v, embs_v, num_v, w_v, b_v, proj_v, sem):
        wid = lax.axis_index("s") * NC + lax.axis_index("c")
        base = wid * BPW

        pltpu.sync_copy(gidx_hbm.at[:, pl.ds(base, BPW)], idx_v)

        gathers = []
        for f in range(N_CAT):
            gathers.append(
                pltpu.async_copy(tab_hbm.at[idx_v.at[f]], embs_v.at[f], sem))

        pltpu.sync_copy(num_hbm.at[pl.ds(base * F_NUM, BPW * F_NUM)],
                        num_v.at[pl.ds(0, BPW * F_NUM)])
        pltpu.sync_copy(w_hbm, w_v)
        pltpu.sync_copy(b_hbm, b_v)

        w_lo = [w_v[k, pl.ds(0, 16)] for k in range(F_NUM)]
        w_hi = [w_v[k, pl.ds(16, 16)] for k in range(F_NUM)]
        b_lo = b_v[pl.ds(0, 16)]
        b_hi = b_v[pl.ds(16, 16)]

        def row_body(r, carry):
            v = num_v[pl.ds(r * F_NUM, 16)]
            a0 = b_lo
            a1 = b_hi
            for k in range(F_NUM):
                x = v[k]
                a0 = a0 + x * w_lo[k]
                a1 = a1 + x * w_hi[k]
            proj_v[r, pl.ds(0, 16)] = a0
            proj_v[r, pl.ds(16, 16)] = a1
            return carry

        lax.fori_loop(0, BPW, row_body, 0)

        pltpu.sync_copy(proj_v, out_hbm.at[pl.ds(base, BPW), pl.ds(0, P)])

        for f in range(N_CAT):
            gathers[f].wait()
            pltpu.sync_copy(
                embs_v.at[f],
                out_hbm.at[pl.ds(base, BPW), pl.ds(P + f * E, E)])

    return enc(gidx, numeric, table_flat, W, b)


def kernel(numeric, idx, table, W, b):
    gidx = idx.astype(jnp.int32) + (
        jnp.arange(N_CAT, dtype=jnp.int32) * VOCAB)[:, None]
    table_flat = table.reshape(N_CAT * VOCAB, E)
    return _sc_encoder(gidx, numeric.reshape(-1), table_flat, W, b)
